# baseline (speedup 1.0000x reference)
import jax
import jax.numpy as jnp
from jax import lax
from jax.experimental import pallas as pl
from jax.experimental.pallas import tpu as pltpu
from jax.experimental.pallas import tpu_sc as plsc

N = 10000
E = 320000
D = 128
NC = 2
NS = 16
NW = NC * NS
EPW = E // NW
K = 128
NCHUNK = 80
RING = 16
NBLK = NCHUNK // RING
NPAD = 10240
ZPT = NPAD // NS
RPT = NPAD // NS
BN = 1000

_mesh = plsc.VectorSubcoreMesh(core_axis_name="c", subcore_axis_name="s")


def _deg_body(src_hbm, dst_hbm, out_hbm, src_v, dst_v, ones_v, zero_v,
              acc_o, acc_i):
    c = lax.axis_index("c")
    s = lax.axis_index("s")
    wid = c * NS + s
    for i in range(ZPT // 16):
        zero_v[pl.ds(i * 16, 16)] = jnp.zeros((16,), jnp.float32)
    for i in range(K // 16):
        ones_v[pl.ds(i * 16, 16)] = jnp.ones((16,), jnp.float32)
    pltpu.sync_copy(zero_v, acc_o.at[pl.ds(s * ZPT, ZPT)])
    pltpu.sync_copy(zero_v, acc_i.at[pl.ds(s * ZPT, ZPT)])
    pltpu.sync_copy(src_hbm.at[wid], src_v)
    pltpu.sync_copy(dst_hbm.at[wid], dst_v)
    plsc.subcore_barrier()

    @pl.loop(0, NCHUNK)
    def _chunk(j):
        pltpu.sync_copy(ones_v, acc_o.at[src_v.at[j]], add=True)
        pltpu.sync_copy(ones_v, acc_i.at[dst_v.at[j]], add=True)

    plsc.subcore_barrier()
    pltpu.sync_copy(acc_o.at[pl.ds(s * ZPT, ZPT)],
                    out_hbm.at[c, 0, pl.ds(s * ZPT, ZPT)])
    pltpu.sync_copy(acc_i.at[pl.ds(s * ZPT, ZPT)],
                    out_hbm.at[c, 1, pl.ds(s * ZPT, ZPT)])


_deg_kernel = pl.kernel(
    _deg_body,
    out_type=jax.ShapeDtypeStruct((NC, 2, NPAD), jnp.float32),
    mesh=_mesh,
    scratch_types=[
        pltpu.VMEM((NCHUNK, K), jnp.int32),
        pltpu.VMEM((NCHUNK, K), jnp.int32),
        pltpu.VMEM((K,), jnp.float32),
        pltpu.VMEM((ZPT,), jnp.float32),
        pltpu.VMEM_SHARED((NPAD + 16,), jnp.float32),
        pltpu.VMEM_SHARED((NPAD + 16,), jnp.float32),
    ],
)


HALF = NPAD // 2
ACC_R = HALF + 128
QROWS = 832
KQ = 64
SCAN_B = 16
EPT = 327680 // NS


def _agg_body(x_hbm, pk_hbm, out_hbm, qs0, qs1, sbuf, sidx0, didx0, sidx1,
              didx1, rows0, rows1, sem0, sem1, xstage, acc):
    c = lax.axis_index("c")
    s = lax.axis_index("s")
    is_hi = c > 0

    @pl.loop(0, KQ)
    def _zrow(j):
        for l in range(D // 16):
            rows0[j, pl.ds(l * 16, 16)] = jnp.zeros((16,), jnp.float32)

    z0 = s * (ACC_R // NS)
    for m in range(5):
        pltpu.sync_copy(rows0, acc.at[pl.ds(z0 + m * KQ, KQ)])
    pltpu.sync_copy(rows0.at[pl.ds(0, 8)], acc.at[pl.ds(z0 + 320, 8)])

    dump = (c + 1) * HALF
    dum0 = jnp.broadcast_to(dump * 16384, (16,)).astype(jnp.int32)
    dum1 = jnp.broadcast_to(dump * 16384 + HALF, (16,)).astype(jnp.int32)

    @pl.loop(0, QROWS)
    def _fill(r):
        qs0[pl.ds(r * 16, 16)] = dum0
        qs1[pl.ds(r * 16, 16)] = dum1

    lane = lax.iota(jnp.int32, 16)
    zero16 = jnp.zeros((16,), jnp.int32)

    @pl.loop(0, EPT // D // SCAN_B, init_carry=(zero16, zero16))
    def _scan_blk(b, cur):
        cur0, cur1 = cur
        pltpu.sync_copy(pk_hbm.at[s].at[pl.ds(b * SCAN_B, SCAN_B)], sbuf)
        for r in range(SCAN_B):
            for l in range(D // 16):
                v = sbuf[r, pl.ds(l * 16, 16)]
                srcg = jnp.bitwise_and(v, 16383)
                dstg = lax.shift_right_logical(v, 14)
                dlo = lax.shift_right_logical(dstg - HALF, 31)
                slo = lax.shift_right_logical(srcg - HALF, 31)
                kept = jnp.bitwise_xor(dlo, c)
                k0 = kept * slo
                k1 = kept - k0
                i0 = jnp.minimum(cur0, QROWS - 1) * 16 * k0 \
                    + (1 - k0) * (QROWS * 16) + lane
                i1 = jnp.minimum(cur1, QROWS - 1) * 16 * k1 \
                    + (1 - k1) * (QROWS * 16) + lane
                plsc.store_scatter(qs0, [i0], v)
                plsc.store_scatter(qs1, [i1], v)
                cur0 = cur0 + k0
                cur1 = cur1 + k1
        return cur0, cur1

    cur0, cur1 = _scan_blk

    def _vmax(vec):
        rmax = jnp.int32(0)
        cand = jnp.ones((16,), jnp.int32)
        for k in range(14, -1, -1):
            bitk = jnp.bitwise_and(lax.shift_right_logical(vec, k), 1)
            t = jnp.any(jnp.logical_and(bitk == 1, cand == 1)).astype(jnp.int32)
            rmax = rmax * 2 + t
            cand = cand * (bitk * t + (1 - t))
        return rmax

    np0 = jnp.minimum(lax.shift_right_logical(_vmax(cur0) + 7, 3), QROWS // 8)
    np1 = jnp.minimum(lax.shift_right_logical(_vmax(cur1) + 7, 3), QROWS // 8)

    def _run_pass(qs, npair, p):
        pltpu.sync_copy(x_hbm.at[pl.ds(p * HALF + s * 320, 320)],
                        xstage.at[pl.ds(s * 320, 320)])
        plsc.subcore_barrier()

        @pl.loop(0, npair)
        def _pair(j):
            base = j * 128
            for off, sb, db in ((0, sidx0, didx0), (KQ, sidx1, didx1)):
                for l in range(KQ // 16):
                    v = qs[pl.ds(base + off + l * 16, 16)]
                    sb[pl.ds(l * 16, 16)] = jnp.bitwise_and(v, 16383) - p * HALF
                    db[pl.ds(l * 16, 16)] = lax.shift_right_logical(v, 14) - c * HALF
            g0 = pltpu.async_copy(xstage.at[sidx0], rows0, sem0)
            g1 = pltpu.async_copy(xstage.at[sidx1], rows1, sem1)
            g0.wait()
            pltpu.sync_copy(rows0, acc.at[didx0], add=True)
            g1.wait()
            pltpu.sync_copy(rows1, acc.at[didx1], add=True)

        plsc.subcore_barrier()

    _run_pass(qs0, np0, 0)
    _run_pass(qs1, np1, 1)

    pltpu.sync_copy(acc.at[pl.ds(s * 320, 320)],
                    out_hbm.at[pl.ds(c * HALF + s * 320, 320)])


_agg_kernel = pl.kernel(
    _agg_body,
    out_type=jax.ShapeDtypeStruct((NPAD, D), jnp.float32),
    mesh=_mesh,
    compiler_params=pltpu.CompilerParams(needs_layout_passes=False),
    scratch_types=[
        pltpu.VMEM((QROWS * 16 + 16,), jnp.int32),
        pltpu.VMEM((QROWS * 16 + 16,), jnp.int32),
        pltpu.VMEM((SCAN_B, D), jnp.int32),
        pltpu.VMEM((KQ,), jnp.int32),
        pltpu.VMEM((KQ,), jnp.int32),
        pltpu.VMEM((KQ,), jnp.int32),
        pltpu.VMEM((KQ,), jnp.int32),
        pltpu.VMEM((KQ, D), jnp.float32),
        pltpu.VMEM((KQ, D), jnp.float32),
        pltpu.SemaphoreType.DMA,
        pltpu.SemaphoreType.DMA,
        pltpu.VMEM_SHARED((HALF, D), jnp.float32),
        pltpu.VMEM_SHARED((ACC_R, D), jnp.float32),
    ],
)


def _norms_body(dego_ref, degi_ref, feat_ref, x0s_ref, ns_ref, nd_ref):
    d_o = dego_ref[0] + dego_ref[1]
    d_i = degi_ref[0] + degi_ref[1]
    ns = lax.rsqrt(jnp.maximum(d_o, 1.0))
    nd = lax.rsqrt(jnp.maximum(d_i, 1.0))
    ns_ref[...] = ns
    nd_ref[...] = nd
    x0s_ref[...] = feat_ref[...] * ns


def _layer_body(agg_ref, w_ref, b_ref, nd_ref, ns_ref, y_ref):
    a = agg_ref[...]
    h = jnp.dot(a, w_ref[...], preferred_element_type=jnp.float32)
    h = h * nd_ref[...] + b_ref[...]
    y_ref[...] = jnp.maximum(h, 0.0) * ns_ref[...]


def _final_body(agg_ref, w_ref, b_ref, nd_ref, out_ref):
    i = pl.program_id(0)
    a = agg_ref[...]
    h = jnp.dot(a, w_ref[...], preferred_element_type=jnp.float32)
    h = h * nd_ref[...] + b_ref[...]
    h = jnp.maximum(h, 0.0)
    part = jnp.sum(h, axis=0, keepdims=True) * (1.0 / N)

    @pl.when(i == 0)
    def _():
        out_ref[...] = jnp.zeros_like(out_ref)

    out_ref[...] += part


def _pad_edges(idx, fill):
    w = idx.reshape(NW, EPW)
    pad = jnp.full((NW, NCHUNK * K - EPW), fill, dtype=idx.dtype)
    return jnp.concatenate([w, pad], axis=1).reshape(NW, NCHUNK, K)


def kernel(features, edge_index, W1, b1, W2, b2):
    src_d = _pad_edges(edge_index[0], N)
    dst = _pad_edges(edge_index[1], N)
    packed = edge_index[0] + edge_index[1] * 16384
    pad = jnp.full((NS * EPT - E,), NPAD * 16384, dtype=jnp.int32)
    packed = jnp.concatenate([packed, pad]).reshape(NS, EPT // D, D)
    b1 = b1.reshape(1, D)
    b2 = b2.reshape(1, D)

    degp = _deg_kernel(src_d, dst)
    deg_o = degp[:, 0, :N, None]
    deg_i = degp[:, 1, :N, None]

    grid = N // BN
    x0s, ns, nd = pl.pallas_call(
        _norms_body,
        grid=(grid,),
        in_specs=[
            pl.BlockSpec((2, BN, 1), lambda i: (0, i, 0)),
            pl.BlockSpec((2, BN, 1), lambda i: (0, i, 0)),
            pl.BlockSpec((BN, D), lambda i: (i, 0)),
        ],
        out_specs=[
            pl.BlockSpec((BN, D), lambda i: (i, 0)),
            pl.BlockSpec((BN, 1), lambda i: (i, 0)),
            pl.BlockSpec((BN, 1), lambda i: (i, 0)),
        ],
        out_shape=[
            jax.ShapeDtypeStruct((NPAD, D), jnp.float32),
            jax.ShapeDtypeStruct((N, 1), jnp.float32),
            jax.ShapeDtypeStruct((N, 1), jnp.float32),
        ],
    )(deg_o, deg_i, features)

    agg1 = _agg_kernel(x0s, packed)

    y1s = pl.pallas_call(
        _layer_body,
        grid=(grid,),
        in_specs=[
            pl.BlockSpec((BN, D), lambda i: (i, 0)),
            pl.BlockSpec((D, D), lambda i: (0, 0)),
            pl.BlockSpec((1, D), lambda i: (0, 0)),
            pl.BlockSpec((BN, 1), lambda i: (i, 0)),
            pl.BlockSpec((BN, 1), lambda i: (i, 0)),
        ],
        out_specs=pl.BlockSpec((BN, D), lambda i: (i, 0)),
        out_shape=jax.ShapeDtypeStruct((NPAD, D), jnp.float32),
    )(agg1, W1, b1, nd, ns)

    agg2 = _agg_kernel(y1s, packed)

    hg = pl.pallas_call(
        _final_body,
        grid=(grid,),
        in_specs=[
            pl.BlockSpec((BN, D), lambda i: (i, 0)),
            pl.BlockSpec((D, D), lambda i: (0, 0)),
            pl.BlockSpec((1, D), lambda i: (0, 0)),
            pl.BlockSpec((BN, 1), lambda i: (i, 0)),
        ],
        out_specs=pl.BlockSpec((1, D), lambda i: (0, 0)),
        out_shape=jax.ShapeDtypeStruct((1, D), jnp.float32),
    )(agg2, W2, b2, nd)

    return hg


# speedup vs baseline: 1.0138x; 1.0138x over previous
import jax
import jax.numpy as jnp
from jax import lax
from jax.experimental import pallas as pl
from jax.experimental.pallas import tpu as pltpu
from jax.experimental.pallas import tpu_sc as plsc

N = 10000
E = 320000
D = 128
NC = 2
NS = 16
NW = NC * NS
EPW = E // NW
K = 128
NCHUNK = 80
RING = 16
NBLK = NCHUNK // RING
NPAD = 10240
ZPT = NPAD // NS
RPT = NPAD // NS
BN = 1000

_mesh = plsc.VectorSubcoreMesh(core_axis_name="c", subcore_axis_name="s")


def _deg_body(src_hbm, dst_hbm, out_hbm, src_v, dst_v, ones_v, zero_v,
              acc_o, acc_i):
    c = lax.axis_index("c")
    s = lax.axis_index("s")
    wid = c * NS + s
    for i in range(ZPT // 16):
        zero_v[pl.ds(i * 16, 16)] = jnp.zeros((16,), jnp.float32)
    for i in range(K // 16):
        ones_v[pl.ds(i * 16, 16)] = jnp.ones((16,), jnp.float32)
    pltpu.sync_copy(zero_v, acc_o.at[pl.ds(s * ZPT, ZPT)])
    pltpu.sync_copy(zero_v, acc_i.at[pl.ds(s * ZPT, ZPT)])
    pltpu.sync_copy(src_hbm.at[wid], src_v)
    pltpu.sync_copy(dst_hbm.at[wid], dst_v)
    plsc.subcore_barrier()

    @pl.loop(0, NCHUNK)
    def _chunk(j):
        pltpu.sync_copy(ones_v, acc_o.at[src_v.at[j]], add=True)
        pltpu.sync_copy(ones_v, acc_i.at[dst_v.at[j]], add=True)

    plsc.subcore_barrier()
    pltpu.sync_copy(acc_o.at[pl.ds(s * ZPT, ZPT)],
                    out_hbm.at[c, 0, pl.ds(s * ZPT, ZPT)])
    pltpu.sync_copy(acc_i.at[pl.ds(s * ZPT, ZPT)],
                    out_hbm.at[c, 1, pl.ds(s * ZPT, ZPT)])


_deg_kernel = pl.kernel(
    _deg_body,
    out_type=jax.ShapeDtypeStruct((NC, 2, NPAD), jnp.float32),
    mesh=_mesh,
    scratch_types=[
        pltpu.VMEM((NCHUNK, K), jnp.int32),
        pltpu.VMEM((NCHUNK, K), jnp.int32),
        pltpu.VMEM((K,), jnp.float32),
        pltpu.VMEM((ZPT,), jnp.float32),
        pltpu.VMEM_SHARED((NPAD + 16,), jnp.float32),
        pltpu.VMEM_SHARED((NPAD + 16,), jnp.float32),
    ],
)


HALF = NPAD // 2
ACC_R = HALF + 128
QROWS = 832
KQ = 64
SCAN_B = 16
EPT = 327680 // NS


def _agg_body(x_hbm, pk_hbm, out_hbm, qs0, qs1, sbuf, sidx0, didx0, sidx1,
              didx1, rows0, rows1, sem0, sem1, sem2, sem3, xstage, acc):
    c = lax.axis_index("c")
    s = lax.axis_index("s")
    is_hi = c > 0

    @pl.loop(0, KQ)
    def _zrow(j):
        for l in range(D // 16):
            rows0[j, pl.ds(l * 16, 16)] = jnp.zeros((16,), jnp.float32)

    z0 = s * (ACC_R // NS)
    for m in range(5):
        pltpu.sync_copy(rows0, acc.at[pl.ds(z0 + m * KQ, KQ)])
    pltpu.sync_copy(rows0.at[pl.ds(0, 8)], acc.at[pl.ds(z0 + 320, 8)])

    dump = (c + 1) * HALF
    dum0 = jnp.broadcast_to(dump * 16384, (16,)).astype(jnp.int32)
    dum1 = jnp.broadcast_to(dump * 16384 + HALF, (16,)).astype(jnp.int32)

    @pl.loop(0, QROWS)
    def _fill(r):
        qs0[pl.ds(r * 16, 16)] = dum0
        qs1[pl.ds(r * 16, 16)] = dum1

    lane = lax.iota(jnp.int32, 16)
    zero16 = jnp.zeros((16,), jnp.int32)

    @pl.loop(0, EPT // D // SCAN_B, init_carry=(zero16, zero16))
    def _scan_blk(b, cur):
        cur0, cur1 = cur
        pltpu.sync_copy(pk_hbm.at[s].at[pl.ds(b * SCAN_B, SCAN_B)], sbuf)
        for r in range(SCAN_B):
            for l in range(D // 16):
                v = sbuf[r, pl.ds(l * 16, 16)]
                srcg = jnp.bitwise_and(v, 16383)
                dstg = lax.shift_right_logical(v, 14)
                dlo = lax.shift_right_logical(dstg - HALF, 31)
                slo = lax.shift_right_logical(srcg - HALF, 31)
                kept = jnp.bitwise_xor(dlo, c)
                k0 = kept * slo
                k1 = kept - k0
                i0 = jnp.minimum(cur0, QROWS - 1) * 16 * k0 \
                    + (1 - k0) * (QROWS * 16) + lane
                i1 = jnp.minimum(cur1, QROWS - 1) * 16 * k1 \
                    + (1 - k1) * (QROWS * 16) + lane
                plsc.store_scatter(qs0, [i0], v)
                plsc.store_scatter(qs1, [i1], v)
                cur0 = cur0 + k0
                cur1 = cur1 + k1
        return cur0, cur1

    cur0, cur1 = _scan_blk

    def _vmax(vec):
        rmax = jnp.int32(0)
        cand = jnp.ones((16,), jnp.int32)
        for k in range(14, -1, -1):
            bitk = jnp.bitwise_and(lax.shift_right_logical(vec, k), 1)
            t = jnp.any(jnp.logical_and(bitk == 1, cand == 1)).astype(jnp.int32)
            rmax = rmax * 2 + t
            cand = cand * (bitk * t + (1 - t))
        return rmax

    np0 = jnp.minimum(lax.shift_right_logical(_vmax(cur0) + 7, 3), QROWS // 8)
    np1 = jnp.minimum(lax.shift_right_logical(_vmax(cur1) + 7, 3), QROWS // 8)

    def _run_pass(qs, npair, p):
        pltpu.sync_copy(x_hbm.at[pl.ds(p * HALF + s * 320, 320)],
                        xstage.at[pl.ds(s * 320, 320)])
        plsc.subcore_barrier()

        @pl.loop(0, npair)
        def _pair(j):
            @pl.when(j > 0)
            def _():
                pltpu.make_async_copy(rows0, acc.at[didx0], sem2).wait()
                pltpu.make_async_copy(rows1, acc.at[didx1], sem3).wait()

            base = j * 128
            for off, sb, db in ((0, sidx0, didx0), (KQ, sidx1, didx1)):
                for l in range(KQ // 16):
                    v = qs[pl.ds(base + off + l * 16, 16)]
                    sb[pl.ds(l * 16, 16)] = jnp.bitwise_and(v, 16383) - p * HALF
                    db[pl.ds(l * 16, 16)] = lax.shift_right_logical(v, 14) - c * HALF
            g0 = pltpu.async_copy(xstage.at[sidx0], rows0, sem0)
            g1 = pltpu.async_copy(xstage.at[sidx1], rows1, sem1)
            g0.wait()
            pltpu.async_copy(rows0, acc.at[didx0], sem2, add=True)
            g1.wait()
            pltpu.async_copy(rows1, acc.at[didx1], sem3, add=True)

        @pl.when(npair > 0)
        def _():
            pltpu.make_async_copy(rows0, acc.at[didx0], sem2).wait()
            pltpu.make_async_copy(rows1, acc.at[didx1], sem3).wait()

        plsc.subcore_barrier()

    _run_pass(qs0, np0, 0)
    _run_pass(qs1, np1, 1)

    pltpu.sync_copy(acc.at[pl.ds(s * 320, 320)],
                    out_hbm.at[pl.ds(c * HALF + s * 320, 320)])


_agg_kernel = pl.kernel(
    _agg_body,
    out_type=jax.ShapeDtypeStruct((NPAD, D), jnp.float32),
    mesh=_mesh,
    compiler_params=pltpu.CompilerParams(needs_layout_passes=False),
    scratch_types=[
        pltpu.VMEM((QROWS * 16 + 16,), jnp.int32),
        pltpu.VMEM((QROWS * 16 + 16,), jnp.int32),
        pltpu.VMEM((SCAN_B, D), jnp.int32),
        pltpu.VMEM((KQ,), jnp.int32),
        pltpu.VMEM((KQ,), jnp.int32),
        pltpu.VMEM((KQ,), jnp.int32),
        pltpu.VMEM((KQ,), jnp.int32),
        pltpu.VMEM((KQ, D), jnp.float32),
        pltpu.VMEM((KQ, D), jnp.float32),
        pltpu.SemaphoreType.DMA,
        pltpu.SemaphoreType.DMA,
        pltpu.SemaphoreType.DMA,
        pltpu.SemaphoreType.DMA,
        pltpu.VMEM_SHARED((HALF, D), jnp.float32),
        pltpu.VMEM_SHARED((ACC_R, D), jnp.float32),
    ],
)


def _norms_body(dego_ref, degi_ref, feat_ref, x0s_ref, ns_ref, nd_ref):
    d_o = dego_ref[0] + dego_ref[1]
    d_i = degi_ref[0] + degi_ref[1]
    ns = lax.rsqrt(jnp.maximum(d_o, 1.0))
    nd = lax.rsqrt(jnp.maximum(d_i, 1.0))
    ns_ref[...] = ns
    nd_ref[...] = nd
    x0s_ref[...] = feat_ref[...] * ns


def _layer_body(agg_ref, w_ref, b_ref, nd_ref, ns_ref, y_ref):
    a = agg_ref[...]
    h = jnp.dot(a, w_ref[...], preferred_element_type=jnp.float32)
    h = h * nd_ref[...] + b_ref[...]
    y_ref[...] = jnp.maximum(h, 0.0) * ns_ref[...]


def _final_body(agg_ref, w_ref, b_ref, nd_ref, out_ref):
    i = pl.program_id(0)
    a = agg_ref[...]
    h = jnp.dot(a, w_ref[...], preferred_element_type=jnp.float32)
    h = h * nd_ref[...] + b_ref[...]
    h = jnp.maximum(h, 0.0)
    part = jnp.sum(h, axis=0, keepdims=True) * (1.0 / N)

    @pl.when(i == 0)
    def _():
        out_ref[...] = jnp.zeros_like(out_ref)

    out_ref[...] += part


def _pad_edges(idx, fill):
    w = idx.reshape(NW, EPW)
    pad = jnp.full((NW, NCHUNK * K - EPW), fill, dtype=idx.dtype)
    return jnp.concatenate([w, pad], axis=1).reshape(NW, NCHUNK, K)


def kernel(features, edge_index, W1, b1, W2, b2):
    src_d = _pad_edges(edge_index[0], N)
    dst = _pad_edges(edge_index[1], N)
    packed = edge_index[0] + edge_index[1] * 16384
    pad = jnp.full((NS * EPT - E,), NPAD * 16384, dtype=jnp.int32)
    packed = jnp.concatenate([packed, pad]).reshape(NS, EPT // D, D)
    b1 = b1.reshape(1, D)
    b2 = b2.reshape(1, D)

    degp = _deg_kernel(src_d, dst)
    deg_o = degp[:, 0, :N, None]
    deg_i = degp[:, 1, :N, None]

    grid = N // BN
    x0s, ns, nd = pl.pallas_call(
        _norms_body,
        grid=(grid,),
        in_specs=[
            pl.BlockSpec((2, BN, 1), lambda i: (0, i, 0)),
            pl.BlockSpec((2, BN, 1), lambda i: (0, i, 0)),
            pl.BlockSpec((BN, D), lambda i: (i, 0)),
        ],
        out_specs=[
            pl.BlockSpec((BN, D), lambda i: (i, 0)),
            pl.BlockSpec((BN, 1), lambda i: (i, 0)),
            pl.BlockSpec((BN, 1), lambda i: (i, 0)),
        ],
        out_shape=[
            jax.ShapeDtypeStruct((NPAD, D), jnp.float32),
            jax.ShapeDtypeStruct((N, 1), jnp.float32),
            jax.ShapeDtypeStruct((N, 1), jnp.float32),
        ],
    )(deg_o, deg_i, features)

    agg1 = _agg_kernel(x0s, packed)

    y1s = pl.pallas_call(
        _layer_body,
        grid=(grid,),
        in_specs=[
            pl.BlockSpec((BN, D), lambda i: (i, 0)),
            pl.BlockSpec((D, D), lambda i: (0, 0)),
            pl.BlockSpec((1, D), lambda i: (0, 0)),
            pl.BlockSpec((BN, 1), lambda i: (i, 0)),
            pl.BlockSpec((BN, 1), lambda i: (i, 0)),
        ],
        out_specs=pl.BlockSpec((BN, D), lambda i: (i, 0)),
        out_shape=jax.ShapeDtypeStruct((NPAD, D), jnp.float32),
    )(agg1, W1, b1, nd, ns)

    agg2 = _agg_kernel(y1s, packed)

    hg = pl.pallas_call(
        _final_body,
        grid=(grid,),
        in_specs=[
            pl.BlockSpec((BN, D), lambda i: (i, 0)),
            pl.BlockSpec((D, D), lambda i: (0, 0)),
            pl.BlockSpec((1, D), lambda i: (0, 0)),
            pl.BlockSpec((BN, 1), lambda i: (i, 0)),
        ],
        out_specs=pl.BlockSpec((1, D), lambda i: (0, 0)),
        out_shape=jax.ShapeDtypeStruct((1, D), jnp.float32),
    )(agg2, W2, b2, nd)

    return hg


# speedup vs baseline: 1.1504x; 1.1347x over previous
import jax
import jax.numpy as jnp
from jax import lax
from jax.experimental import pallas as pl
from jax.experimental.pallas import tpu as pltpu
from jax.experimental.pallas import tpu_sc as plsc

N = 10000
E = 320000
D = 128
NC = 2
NS = 16
NW = NC * NS
EPW = E // NW
K = 128
NCHUNK = 80
RING = 16
NBLK = NCHUNK // RING
NPAD = 10240
ZPT = NPAD // NS
RPT = NPAD // NS
BN = 1000

_mesh = plsc.VectorSubcoreMesh(core_axis_name="c", subcore_axis_name="s")


def _deg_body(src_hbm, dst_hbm, out_hbm, src_v, dst_v, ones_v, zero_v,
              acc_o, acc_i):
    c = lax.axis_index("c")
    s = lax.axis_index("s")
    wid = c * NS + s
    for i in range(ZPT // 16):
        zero_v[pl.ds(i * 16, 16)] = jnp.zeros((16,), jnp.float32)
    for i in range(K // 16):
        ones_v[pl.ds(i * 16, 16)] = jnp.ones((16,), jnp.float32)
    pltpu.sync_copy(zero_v, acc_o.at[pl.ds(s * ZPT, ZPT)])
    pltpu.sync_copy(zero_v, acc_i.at[pl.ds(s * ZPT, ZPT)])
    pltpu.sync_copy(src_hbm.at[wid], src_v)
    pltpu.sync_copy(dst_hbm.at[wid], dst_v)
    plsc.subcore_barrier()

    @pl.loop(0, NCHUNK)
    def _chunk(j):
        pltpu.sync_copy(ones_v, acc_o.at[src_v.at[j]], add=True)
        pltpu.sync_copy(ones_v, acc_i.at[dst_v.at[j]], add=True)

    plsc.subcore_barrier()
    pltpu.sync_copy(acc_o.at[pl.ds(s * ZPT, ZPT)],
                    out_hbm.at[c, 0, pl.ds(s * ZPT, ZPT)])
    pltpu.sync_copy(acc_i.at[pl.ds(s * ZPT, ZPT)],
                    out_hbm.at[c, 1, pl.ds(s * ZPT, ZPT)])


_deg_kernel = pl.kernel(
    _deg_body,
    out_type=jax.ShapeDtypeStruct((NC, 2, NPAD), jnp.float32),
    mesh=_mesh,
    scratch_types=[
        pltpu.VMEM((NCHUNK, K), jnp.int32),
        pltpu.VMEM((NCHUNK, K), jnp.int32),
        pltpu.VMEM((K,), jnp.float32),
        pltpu.VMEM((ZPT,), jnp.float32),
        pltpu.VMEM_SHARED((NPAD + 16,), jnp.float32),
        pltpu.VMEM_SHARED((NPAD + 16,), jnp.float32),
    ],
)


HALF = NPAD // 2
ACC_R = HALF + 128
QROWS = 832
KQ = 64
SCAN_B = 16
EPT = 327680 // NS


def _agg_body(x_hbm, pk_hbm, out_hbm, qs0, qs1, sbuf, sidx0, didx0, sidx1,
              didx1, rows0, rows1, sem0, sem1, sem2, sem3, xstage, acc):
    c = lax.axis_index("c")
    s = lax.axis_index("s")
    is_hi = c > 0

    @pl.loop(0, KQ)
    def _zrow(j):
        for l in range(D // 16):
            rows0[j, pl.ds(l * 16, 16)] = jnp.zeros((16,), jnp.float32)

    z0 = s * (ACC_R // NS)
    for m in range(5):
        pltpu.sync_copy(rows0, acc.at[pl.ds(z0 + m * KQ, KQ)])
    pltpu.sync_copy(rows0.at[pl.ds(0, 8)], acc.at[pl.ds(z0 + 320, 8)])

    dump = (c + 1) * HALF
    dum0 = jnp.broadcast_to(dump * 16384, (16,)).astype(jnp.int32)
    dum1 = jnp.broadcast_to(dump * 16384 + HALF, (16,)).astype(jnp.int32)

    @pl.loop(0, QROWS)
    def _fill(r):
        qs0[pl.ds(r * 16, 16)] = dum0
        qs1[pl.ds(r * 16, 16)] = dum1

    lane = lax.iota(jnp.int32, 16)
    zero16 = jnp.zeros((16,), jnp.int32)

    @pl.loop(0, EPT // D // SCAN_B, init_carry=(zero16, zero16))
    def _scan_blk(b, cur):
        cur0, cur1 = cur
        pltpu.sync_copy(pk_hbm.at[s].at[pl.ds(b * SCAN_B, SCAN_B)], sbuf)
        for r in range(SCAN_B):
            for l in range(D // 16):
                v = sbuf[r, pl.ds(l * 16, 16)]
                srcg = jnp.bitwise_and(v, 16383)
                dstg = lax.shift_right_logical(v, 14)
                dlo = lax.shift_right_logical(dstg - HALF, 31)
                slo = lax.shift_right_logical(srcg - HALF, 31)
                val = lax.shift_right_logical(dstg - 16000, 31)
                kept = jnp.bitwise_xor(dlo, c) * val
                k0 = kept * slo
                k1 = kept - k0
                i0 = jnp.minimum(cur0, QROWS - 1) * 16 * k0 \
                    + (1 - k0) * (QROWS * 16) + lane
                i1 = jnp.minimum(cur1, QROWS - 1) * 16 * k1 \
                    + (1 - k1) * (QROWS * 16) + lane
                plsc.store_scatter(qs0, [i0], v)
                plsc.store_scatter(qs1, [i1], v)
                cur0 = cur0 + k0
                cur1 = cur1 + k1
        return cur0, cur1

    cur0, cur1 = _scan_blk

    def _vmax(vec):
        rmax = jnp.int32(0)
        cand = jnp.ones((16,), jnp.int32)
        for k in range(14, -1, -1):
            bitk = jnp.bitwise_and(lax.shift_right_logical(vec, k), 1)
            t = jnp.any(jnp.logical_and(bitk == 1, cand == 1)).astype(jnp.int32)
            rmax = rmax * 2 + t
            cand = cand * (bitk * t + (1 - t))
        return rmax

    np0 = jnp.minimum(lax.shift_right_logical(_vmax(cur0) + 7, 3), QROWS // 8)
    np1 = jnp.minimum(lax.shift_right_logical(_vmax(cur1) + 7, 3), QROWS // 8)

    def _run_pass(qs, npair, p):
        pltpu.sync_copy(x_hbm.at[pl.ds(p * HALF + s * 320, 320)],
                        xstage.at[pl.ds(s * 320, 320)])
        plsc.subcore_barrier()

        @pl.loop(0, npair)
        def _pair(j):
            @pl.when(j > 0)
            def _():
                pltpu.make_async_copy(rows0, acc.at[didx0], sem2).wait()
                pltpu.make_async_copy(rows1, acc.at[didx1], sem3).wait()

            base = j * 128
            for off, sb, db in ((0, sidx0, didx0), (KQ, sidx1, didx1)):
                for l in range(KQ // 16):
                    v = qs[pl.ds(base + off + l * 16, 16)]
                    sb[pl.ds(l * 16, 16)] = jnp.bitwise_and(v, 16383) - p * HALF
                    db[pl.ds(l * 16, 16)] = lax.shift_right_logical(v, 14) - c * HALF
            g0 = pltpu.async_copy(xstage.at[sidx0], rows0, sem0)
            g1 = pltpu.async_copy(xstage.at[sidx1], rows1, sem1)
            g0.wait()
            pltpu.async_copy(rows0, acc.at[didx0], sem2, add=True)
            g1.wait()
            pltpu.async_copy(rows1, acc.at[didx1], sem3, add=True)

        @pl.when(npair > 0)
        def _():
            pltpu.make_async_copy(rows0, acc.at[didx0], sem2).wait()
            pltpu.make_async_copy(rows1, acc.at[didx1], sem3).wait()

        plsc.subcore_barrier()

    _run_pass(qs0, np0, 0)
    _run_pass(qs1, np1, 1)

    pltpu.sync_copy(acc.at[pl.ds(s * 320, 320)],
                    out_hbm.at[pl.ds(c * HALF + s * 320, 320)])


_agg_kernel = pl.kernel(
    _agg_body,
    out_type=jax.ShapeDtypeStruct((NPAD, D), jnp.float32),
    mesh=_mesh,
    compiler_params=pltpu.CompilerParams(needs_layout_passes=False),
    scratch_types=[
        pltpu.VMEM((QROWS * 16 + 16,), jnp.int32),
        pltpu.VMEM((QROWS * 16 + 16,), jnp.int32),
        pltpu.VMEM((SCAN_B, D), jnp.int32),
        pltpu.VMEM((KQ,), jnp.int32),
        pltpu.VMEM((KQ,), jnp.int32),
        pltpu.VMEM((KQ,), jnp.int32),
        pltpu.VMEM((KQ,), jnp.int32),
        pltpu.VMEM((KQ, D), jnp.float32),
        pltpu.VMEM((KQ, D), jnp.float32),
        pltpu.SemaphoreType.DMA,
        pltpu.SemaphoreType.DMA,
        pltpu.SemaphoreType.DMA,
        pltpu.SemaphoreType.DMA,
        pltpu.VMEM_SHARED((HALF, D), jnp.float32),
        pltpu.VMEM_SHARED((ACC_R, D), jnp.float32),
    ],
)


def _norms_body(dego_ref, degi_ref, feat_ref, x0s_ref, ns_ref, nd_ref):
    d_o = dego_ref[0] + dego_ref[1]
    d_i = degi_ref[0] + degi_ref[1]
    ns = lax.rsqrt(jnp.maximum(d_o, 1.0))
    nd = lax.rsqrt(jnp.maximum(d_i, 1.0))
    ns_ref[...] = ns
    nd_ref[...] = nd
    x0s_ref[...] = feat_ref[...] * ns


def _layer_body(agg_ref, w_ref, b_ref, nd_ref, ns_ref, y_ref):
    a = agg_ref[...]
    h = jnp.dot(a, w_ref[...], preferred_element_type=jnp.float32)
    h = h * nd_ref[...] + b_ref[...]
    y_ref[...] = jnp.maximum(h, 0.0) * ns_ref[...]


def _final_body(agg_ref, w_ref, b_ref, nd_ref, out_ref):
    i = pl.program_id(0)
    a = agg_ref[...]
    h = jnp.dot(a, w_ref[...], preferred_element_type=jnp.float32)
    h = h * nd_ref[...] + b_ref[...]
    h = jnp.maximum(h, 0.0)
    part = jnp.sum(h, axis=0, keepdims=True) * (1.0 / N)

    @pl.when(i == 0)
    def _():
        out_ref[...] = jnp.zeros_like(out_ref)

    out_ref[...] += part


def _pad_edges(idx, fill):
    w = idx.reshape(NW, EPW)
    pad = jnp.full((NW, NCHUNK * K - EPW), fill, dtype=idx.dtype)
    return jnp.concatenate([w, pad], axis=1).reshape(NW, NCHUNK, K)


def kernel(features, edge_index, W1, b1, W2, b2):
    src_d = _pad_edges(edge_index[0], N)
    dst = _pad_edges(edge_index[1], N)
    packed = edge_index[0] + edge_index[1] * 16384
    pad = jnp.full((NS * EPT - E,), 16383 * 16384, dtype=jnp.int32)
    packed = jnp.concatenate([packed, pad]).reshape(NS, EPT // D, D)
    b1 = b1.reshape(1, D)
    b2 = b2.reshape(1, D)

    degp = _deg_kernel(src_d, dst)
    deg_o = degp[:, 0, :N, None]
    deg_i = degp[:, 1, :N, None]

    grid = N // BN
    x0s, ns, nd = pl.pallas_call(
        _norms_body,
        grid=(grid,),
        in_specs=[
            pl.BlockSpec((2, BN, 1), lambda i: (0, i, 0)),
            pl.BlockSpec((2, BN, 1), lambda i: (0, i, 0)),
            pl.BlockSpec((BN, D), lambda i: (i, 0)),
        ],
        out_specs=[
            pl.BlockSpec((BN, D), lambda i: (i, 0)),
            pl.BlockSpec((BN, 1), lambda i: (i, 0)),
            pl.BlockSpec((BN, 1), lambda i: (i, 0)),
        ],
        out_shape=[
            jax.ShapeDtypeStruct((NPAD, D), jnp.float32),
            jax.ShapeDtypeStruct((N, 1), jnp.float32),
            jax.ShapeDtypeStruct((N, 1), jnp.float32),
        ],
    )(deg_o, deg_i, features)

    agg1 = _agg_kernel(x0s, packed)

    y1s = pl.pallas_call(
        _layer_body,
        grid=(grid,),
        in_specs=[
            pl.BlockSpec((BN, D), lambda i: (i, 0)),
            pl.BlockSpec((D, D), lambda i: (0, 0)),
            pl.BlockSpec((1, D), lambda i: (0, 0)),
            pl.BlockSpec((BN, 1), lambda i: (i, 0)),
            pl.BlockSpec((BN, 1), lambda i: (i, 0)),
        ],
        out_specs=pl.BlockSpec((BN, D), lambda i: (i, 0)),
        out_shape=jax.ShapeDtypeStruct((NPAD, D), jnp.float32),
    )(agg1, W1, b1, nd, ns)

    agg2 = _agg_kernel(y1s, packed)

    hg = pl.pallas_call(
        _final_body,
        grid=(grid,),
        in_specs=[
            pl.BlockSpec((BN, D), lambda i: (i, 0)),
            pl.BlockSpec((D, D), lambda i: (0, 0)),
            pl.BlockSpec((1, D), lambda i: (0, 0)),
            pl.BlockSpec((BN, 1), lambda i: (i, 0)),
        ],
        out_specs=pl.BlockSpec((1, D), lambda i: (0, 0)),
        out_shape=jax.ShapeDtypeStruct((1, D), jnp.float32),
    )(agg2, W2, b2, nd)

    return hg
